# baseline (speedup 1.0000x reference)
import functools

import jax
import jax.numpy as jnp
from jax import lax
from jax.experimental import pallas as pl
from jax.experimental.pallas import tpu as pltpu
from jax.experimental.pallas import tpu_sc as plsc

N = 10000
F = 128
S = 64
E = 320000
BN_EPS = 1e-5

NC = 2
NS = 16
NW = NC * NS
K = 128
C = 79
E_PAD = NW * C * K
N_PAD = 10240
RPT = N_PAD // NS


def _sc_aggregate(h, src_r, dst_r, zeros):
    mesh = plsc.VectorSubcoreMesh(core_axis_name="c", subcore_axis_name="s")

    @functools.partial(
        pl.kernel,
        mesh=mesh,
        out_type=jax.ShapeDtypeStruct((NC, N_PAD, F), jnp.float32),
        scratch_types=[
            pltpu.VMEM((C, K), jnp.int32),
            pltpu.VMEM((C, K), jnp.int32),
            pltpu.VMEM((K, F), jnp.float32),
            pltpu.VMEM_SHARED((N_PAD, F), jnp.float32),
            pltpu.SemaphoreType.DMA,
        ],
    )
    def agg(h_hbm, src_hbm, dst_hbm, z_hbm, out_hbm, src_v, dst_v, rows_v,
            acc_sh, sem):
        cid = lax.axis_index("c")
        sid = lax.axis_index("s")
        wid = sid * NC + cid
        row0 = sid * RPT
        pltpu.sync_copy(z_hbm.at[pl.ds(row0, RPT)], acc_sh.at[pl.ds(row0, RPT)])
        pltpu.sync_copy(src_hbm.at[wid], src_v)
        pltpu.sync_copy(dst_hbm.at[wid], dst_v)
        plsc.subcore_barrier()

        def chunk(ci, carry):
            pltpu.async_copy(h_hbm.at[src_v.at[ci]], rows_v, sem).wait()
            pltpu.sync_copy(rows_v, acc_sh.at[dst_v.at[ci]], add=True)
            return carry

        lax.fori_loop(0, C, chunk, 0)
        plsc.subcore_barrier()
        pltpu.sync_copy(acc_sh.at[pl.ds(row0, RPT)],
                        out_hbm.at[cid, pl.ds(row0, RPT)])

    return agg(h, src_r, dst_r, zeros)


def _mlp_body(h_ref, p_ref, s_ref, w1_ref, b1_ref, w2_ref, b2_ref,
              wo_ref, add_ref, hout_ref, pout_ref):
    z = h_ref[...] * s_ref[...] + (p_ref[0] + p_ref[1])
    z = jnp.dot(z, w1_ref[...], preferred_element_type=jnp.float32) + b1_ref[...]
    z = jnp.maximum(z, 0.0)
    hn = jnp.dot(z, w2_ref[...], preferred_element_type=jnp.float32) + b2_ref[...]
    hn = jnp.maximum(hn, 0.0)
    hout_ref[...] = hn
    pout_ref[...] = jnp.dot(hn, wo_ref[...],
                            preferred_element_type=jnp.float32) + add_ref[...]


_BR = 1000


def _tc_layer(h, pagg, scal_row, w1f, b1f, w2, b2, wo, addin):
    rows3 = lambda i: (0, i, 0)
    rows = lambda i: (i, 0)
    full = lambda i: (0, 0)
    return pl.pallas_call(
        _mlp_body,
        grid=(N // _BR,),
        in_specs=[
            pl.BlockSpec((_BR, F), rows),
            pl.BlockSpec((2, _BR, F), rows3),
            pl.BlockSpec((1, F), full),
            pl.BlockSpec((F, F), full),
            pl.BlockSpec((1, F), full),
            pl.BlockSpec((F, F), full),
            pl.BlockSpec((1, F), full),
            pl.BlockSpec((F, S), full),
            pl.BlockSpec((_BR, S), rows),
        ],
        out_specs=[
            pl.BlockSpec((_BR, F), rows),
            pl.BlockSpec((_BR, S), rows),
        ],
        out_shape=[
            jax.ShapeDtypeStruct((N, F), jnp.float32),
            jax.ShapeDtypeStruct((N, S), jnp.float32),
        ],
    )(h, pagg, scal_row, w1f, b1f, w2, b2, wo, addin)


def kernel(x, edge_index,
           W1_0, b1_0, gamma_0, beta_0, rmean_0, rvar_0, W2_0, b2_0, eps_0,
           W1_1, b1_1, gamma_1, beta_1, rmean_1, rvar_1, W2_1, b2_1, eps_1,
           W1_2, b1_2, gamma_2, beta_2, rmean_2, rvar_2, W2_2, b2_2, eps_2,
           W_out, b_out):
    layers = [
        (W1_0, b1_0, gamma_0, beta_0, rmean_0, rvar_0, W2_0, b2_0, eps_0),
        (W1_1, b1_1, gamma_1, beta_1, rmean_1, rvar_1, W2_1, b2_1, eps_1),
        (W1_2, b1_2, gamma_2, beta_2, rmean_2, rvar_2, W2_2, b2_2, eps_2),
    ]
    pad = E_PAD - E
    src_r = jnp.concatenate(
        [edge_index[0], jnp.zeros((pad,), jnp.int32)]).reshape(NW, C, K)
    dst_r = jnp.concatenate(
        [edge_index[1],
         N + (jnp.arange(pad, dtype=jnp.int32) % (N_PAD - N))]).reshape(NW, C, K)
    zeros = jnp.zeros((N_PAD, F), jnp.float32)

    h = x
    pout = jnp.broadcast_to(b_out[None, :], (N, S))
    for l, (W1, b1, gamma, beta, rmean, rvar, W2, b2, eps) in enumerate(layers):
        s = gamma * lax.rsqrt(rvar + BN_EPS)
        w1f = W1 * s[None, :]
        b1f = ((b1 - rmean) * s + beta)[None, :]
        scal_row = (1.0 + eps) * jnp.ones((1, F), jnp.float32)
        wo = lax.dynamic_slice_in_dim(W_out, l * F, F, axis=0)

        pagg = _sc_aggregate(h, src_r, dst_r, zeros)
        h, pout = _tc_layer(h, pagg, scal_row,
                            w1f, b1f, W2, b2[None, :], wo, pout)
    return pout


# speedup vs baseline: 1.1443x; 1.1443x over previous
import functools

import jax
import jax.numpy as jnp
from jax import lax
from jax.experimental import pallas as pl
from jax.experimental.pallas import tpu as pltpu
from jax.experimental.pallas import tpu_sc as plsc

N = 10000
F = 128
S = 64
E = 320000
BN_EPS = 1e-5

NC = 2
NS = 16
NW = NC * NS
K = 96
C = 105
E_PAD = NW * C * K
N_PAD = 10240
RPT = N_PAD // NS


def _sc_aggregate(h, src_r, dst_r, zeros):
    mesh = plsc.VectorSubcoreMesh(core_axis_name="c", subcore_axis_name="s")

    @functools.partial(
        pl.kernel,
        mesh=mesh,
        out_type=jax.ShapeDtypeStruct((NC, N_PAD, F), jnp.float32),
        scratch_types=[
            pltpu.VMEM((C, K), jnp.int32),
            pltpu.VMEM((C, K), jnp.int32),
            pltpu.VMEM((K, F), jnp.float32),
            pltpu.VMEM_SHARED((N_PAD, F), jnp.float32),
            pltpu.SemaphoreType.DMA,
        ],
    )
    def agg(h_hbm, src_hbm, dst_hbm, z_hbm, out_hbm, src_v, dst_v, rows_v,
            acc_sh, sem):
        cid = lax.axis_index("c")
        sid = lax.axis_index("s")
        wid = sid * NC + cid
        row0 = sid * RPT
        pltpu.sync_copy(z_hbm.at[pl.ds(row0, RPT)], acc_sh.at[pl.ds(row0, RPT)])
        pltpu.sync_copy(src_hbm.at[wid], src_v)
        pltpu.sync_copy(dst_hbm.at[wid], dst_v)
        plsc.subcore_barrier()

        def chunk(ci, carry):
            pltpu.async_copy(h_hbm.at[src_v.at[ci]], rows_v, sem).wait()
            pltpu.sync_copy(rows_v, acc_sh.at[dst_v.at[ci]], add=True)
            return carry

        lax.fori_loop(0, C, chunk, 0)
        plsc.subcore_barrier()
        pltpu.sync_copy(acc_sh.at[pl.ds(row0, RPT)],
                        out_hbm.at[cid, pl.ds(row0, RPT)])

    return agg(h, src_r, dst_r, zeros)


def _mlp_body(h_ref, p_ref, s_ref, w1_ref, b1_ref, w2_ref, b2_ref,
              wo_ref, add_ref, hout_ref, pout_ref):
    z = h_ref[...] * s_ref[...] + (p_ref[0] + p_ref[1])
    z = jnp.dot(z, w1_ref[...], preferred_element_type=jnp.float32) + b1_ref[...]
    z = jnp.maximum(z, 0.0)
    hn = jnp.dot(z, w2_ref[...], preferred_element_type=jnp.float32) + b2_ref[...]
    hn = jnp.maximum(hn, 0.0)
    hout_ref[...] = hn
    pout_ref[...] = jnp.dot(hn, wo_ref[...],
                            preferred_element_type=jnp.float32) + add_ref[...]


_BR = 1000


def _tc_layer(h, pagg, scal_row, w1f, b1f, w2, b2, wo, addin):
    rows3 = lambda i: (0, i, 0)
    rows = lambda i: (i, 0)
    full = lambda i: (0, 0)
    return pl.pallas_call(
        _mlp_body,
        grid=(N // _BR,),
        in_specs=[
            pl.BlockSpec((_BR, F), rows),
            pl.BlockSpec((2, _BR, F), rows3),
            pl.BlockSpec((1, F), full),
            pl.BlockSpec((F, F), full),
            pl.BlockSpec((1, F), full),
            pl.BlockSpec((F, F), full),
            pl.BlockSpec((1, F), full),
            pl.BlockSpec((F, S), full),
            pl.BlockSpec((_BR, S), rows),
        ],
        out_specs=[
            pl.BlockSpec((_BR, F), rows),
            pl.BlockSpec((_BR, S), rows),
        ],
        out_shape=[
            jax.ShapeDtypeStruct((N, F), jnp.float32),
            jax.ShapeDtypeStruct((N, S), jnp.float32),
        ],
    )(h, pagg, scal_row, w1f, b1f, w2, b2, wo, addin)


def kernel(x, edge_index,
           W1_0, b1_0, gamma_0, beta_0, rmean_0, rvar_0, W2_0, b2_0, eps_0,
           W1_1, b1_1, gamma_1, beta_1, rmean_1, rvar_1, W2_1, b2_1, eps_1,
           W1_2, b1_2, gamma_2, beta_2, rmean_2, rvar_2, W2_2, b2_2, eps_2,
           W_out, b_out):
    layers = [
        (W1_0, b1_0, gamma_0, beta_0, rmean_0, rvar_0, W2_0, b2_0, eps_0),
        (W1_1, b1_1, gamma_1, beta_1, rmean_1, rvar_1, W2_1, b2_1, eps_1),
        (W1_2, b1_2, gamma_2, beta_2, rmean_2, rvar_2, W2_2, b2_2, eps_2),
    ]
    pad = E_PAD - E
    src_r = jnp.concatenate(
        [edge_index[0], jnp.zeros((pad,), jnp.int32)]).reshape(NW, C, K)
    dst_r = jnp.concatenate(
        [edge_index[1],
         N + (jnp.arange(pad, dtype=jnp.int32) % (N_PAD - N))]).reshape(NW, C, K)
    zeros = jnp.zeros((N_PAD, F), jnp.float32)

    h = x
    pout = jnp.broadcast_to(b_out[None, :], (N, S))
    for l, (W1, b1, gamma, beta, rmean, rvar, W2, b2, eps) in enumerate(layers):
        s = gamma * lax.rsqrt(rvar + BN_EPS)
        w1f = W1 * s[None, :]
        b1f = ((b1 - rmean) * s + beta)[None, :]
        scal_row = (1.0 + eps) * jnp.ones((1, F), jnp.float32)
        wo = lax.dynamic_slice_in_dim(W_out, l * F, F, axis=0)

        pagg = _sc_aggregate(h, src_r, dst_r, zeros)
        h, pout = _tc_layer(h, pagg, scal_row,
                            w1f, b1f, W2, b2[None, :], wo, pout)
    return pout


# speedup vs baseline: 1.6379x; 1.4314x over previous
import functools

import jax
import jax.numpy as jnp
from jax import lax
from jax.experimental import pallas as pl
from jax.experimental.pallas import tpu as pltpu
from jax.experimental.pallas import tpu_sc as plsc

N = 10000
F = 128
S = 64
E = 320000
BN_EPS = 1e-5

NC = 2
NS = 16
NW = NC * NS
K = 80
C = 125
E_PAD = NW * C * K
N_PAD = 10240
RPT = N_PAD // NS


def _sc_aggregate(h, src_r, dst_r, zeros):
    mesh = plsc.VectorSubcoreMesh(core_axis_name="c", subcore_axis_name="s")

    @functools.partial(
        pl.kernel,
        mesh=mesh,
        out_type=jax.ShapeDtypeStruct((NC, N_PAD, F), jnp.float32),
        scratch_types=[
            pltpu.VMEM((C, K), jnp.int32),
            pltpu.VMEM((C, K), jnp.int32),
            pltpu.VMEM((K, F), jnp.float32),
            pltpu.VMEM_SHARED((N_PAD, F), jnp.float32),
            pltpu.SemaphoreType.DMA,
            pltpu.SemaphoreType.DMA,
        ],
    )
    def agg(h_hbm, src_hbm, dst_hbm, z_hbm, out_hbm, src_v, dst_v, rows_v,
            acc_sh, sem, sz):
        cid = lax.axis_index("c")
        sid = lax.axis_index("s")
        wid = sid * NC + cid
        row0 = sid * RPT
        zcp = pltpu.async_copy(z_hbm.at[pl.ds(row0, RPT)],
                               acc_sh.at[pl.ds(row0, RPT)], sz)
        pltpu.sync_copy(src_hbm.at[wid], src_v)
        pltpu.sync_copy(dst_hbm.at[wid], dst_v)
        zcp.wait()
        plsc.subcore_barrier()

        def chunk(ci, carry):
            pltpu.async_copy(h_hbm.at[src_v.at[ci]], rows_v, sem).wait()
            pltpu.sync_copy(rows_v, acc_sh.at[dst_v.at[ci]], add=True)
            return carry

        lax.fori_loop(0, C, chunk, 0)
        plsc.subcore_barrier()
        pltpu.sync_copy(acc_sh.at[pl.ds(row0, RPT)],
                        out_hbm.at[cid, pl.ds(row0, RPT)])

    return agg(h, src_r, dst_r, zeros)


def _mlp_body(h_ref, p_ref, s_ref, w1_ref, b1_ref, w2_ref, b2_ref,
              wo_ref, add_ref, hout_ref, pout_ref):
    z = h_ref[...] * s_ref[...] + (p_ref[0] + p_ref[1])
    z = jnp.dot(z, w1_ref[...], preferred_element_type=jnp.float32) + b1_ref[...]
    z = jnp.maximum(z, 0.0)
    hn = jnp.dot(z, w2_ref[...], preferred_element_type=jnp.float32) + b2_ref[...]
    hn = jnp.maximum(hn, 0.0)
    hout_ref[...] = hn
    pout_ref[...] = jnp.dot(hn, wo_ref[...],
                            preferred_element_type=jnp.float32) + add_ref[...]


_BR = 1000


def _tc_layer(h, pagg, scal_row, w1f, b1f, w2, b2, wo, addin):
    rows3 = lambda i: (0, i, 0)
    rows = lambda i: (i, 0)
    full = lambda i: (0, 0)
    return pl.pallas_call(
        _mlp_body,
        grid=(N // _BR,),
        in_specs=[
            pl.BlockSpec((_BR, F), rows),
            pl.BlockSpec((2, _BR, F), rows3),
            pl.BlockSpec((1, F), full),
            pl.BlockSpec((F, F), full),
            pl.BlockSpec((1, F), full),
            pl.BlockSpec((F, F), full),
            pl.BlockSpec((1, F), full),
            pl.BlockSpec((F, S), full),
            pl.BlockSpec((_BR, S), rows),
        ],
        out_specs=[
            pl.BlockSpec((_BR, F), rows),
            pl.BlockSpec((_BR, S), rows),
        ],
        out_shape=[
            jax.ShapeDtypeStruct((N, F), jnp.float32),
            jax.ShapeDtypeStruct((N, S), jnp.float32),
        ],
    )(h, pagg, scal_row, w1f, b1f, w2, b2, wo, addin)


def kernel(x, edge_index,
           W1_0, b1_0, gamma_0, beta_0, rmean_0, rvar_0, W2_0, b2_0, eps_0,
           W1_1, b1_1, gamma_1, beta_1, rmean_1, rvar_1, W2_1, b2_1, eps_1,
           W1_2, b1_2, gamma_2, beta_2, rmean_2, rvar_2, W2_2, b2_2, eps_2,
           W_out, b_out):
    layers = [
        (W1_0, b1_0, gamma_0, beta_0, rmean_0, rvar_0, W2_0, b2_0, eps_0),
        (W1_1, b1_1, gamma_1, beta_1, rmean_1, rvar_1, W2_1, b2_1, eps_1),
        (W1_2, b1_2, gamma_2, beta_2, rmean_2, rvar_2, W2_2, b2_2, eps_2),
    ]
    src_r = edge_index[0].reshape(NW, C, K)
    dst_r = edge_index[1].reshape(NW, C, K)
    zeros = jnp.zeros((N_PAD, F), jnp.float32)

    h = x
    pout = jnp.broadcast_to(b_out[None, :], (N, S))
    for l, (W1, b1, gamma, beta, rmean, rvar, W2, b2, eps) in enumerate(layers):
        s = gamma * lax.rsqrt(rvar + BN_EPS)
        w1f = W1 * s[None, :]
        b1f = ((b1 - rmean) * s + beta)[None, :]
        scal_row = (1.0 + eps) * jnp.ones((1, F), jnp.float32)
        wo = lax.dynamic_slice_in_dim(W_out, l * F, F, axis=0)

        pagg = _sc_aggregate(h, src_r, dst_r, zeros)
        h, pout = _tc_layer(h, pagg, scal_row,
                            w1f, b1f, W2, b2[None, :], wo, pout)
    return pout
